# baseline (speedup 1.0000x reference)
import functools

import jax
import jax.numpy as jnp
from jax import lax
from jax.experimental import pallas as pl
from jax.experimental.pallas import tpu as pltpu
from jax.experimental.pallas import tpu_sc as plsc

T = 26
V = 100000
D = 16
B = 4096
G = 20

NC, NS = 2, 16
NW = NC * NS
BW = B // NW
R = BW * G
IC = R // 128

TC_ = 1250
LIN = TC_ + 8 - 2
NCH = V // TC_
NTASK = T * NCH
TPW = NTASK // NW


def _tr_kernel(src_hbm, dst_hbm, in0, in1, out0, out1,
               sin0, sin1, sout0, sout1):
    wid = lax.axis_index("c") * NS + lax.axis_index("s")
    inb, outb, sin, sout = (in0, in1), (out0, out1), (sin0, sin1), (sout0, sout1)

    def task_pos(k):
        tsk = wid * TPW + k
        t = tsk // NCH
        v0 = (tsk % NCH) * TC_
        a0 = (v0 // 8) * 8
        return t, v0, a0

    def fire_in(k, p):
        t, _, a0 = task_pos(k)
        for d in range(D):
            pltpu.async_copy(
                src_hbm.at[pl.ds(t * (V * D) + d * V + a0, LIN)],
                inb[p].at[pl.ds(d * LIN, LIN)], sin[p])

    def wait_in(p):
        for d in range(D):
            pltpu.make_async_copy(
                src_hbm.at[pl.ds(d * LIN, LIN)],
                inb[p].at[pl.ds(d * LIN, LIN)], sin[p]).wait()

    def wait_out(p):
        pltpu.make_async_copy(
            outb[p], dst_hbm.at[pl.ds(0, TC_ * D)], sout[p]).wait()

    def step(k, p, first, last):
        wait_in(p)
        if not first:
            wait_out(p)
        t, v0, a0 = task_pos(k)
        m = v0 - a0
        lanes = lax.iota(jnp.int32, D)

        idx0 = lax.iota(jnp.int32, D) * LIN + m

        def row(v, idx):
            outb[p][pl.ds(v * D, D)] = plsc.load_gather(inb[p], [idx])
            return idx + 1

        lax.fori_loop(0, TC_, row, idx0, unroll=8)
        pltpu.async_copy(
            outb[p], dst_hbm.at[pl.ds(t * (V * D) + v0 * D, TC_ * D)],
            sout[p])
        if not last:
            fire_in(k + 2, p)

    fire_in(0, 0)
    fire_in(1, 1)

    def body(j, _):
        step(2 * j, 0, False, False)
        step(2 * j + 1, 1, False, False)
        return 0

    step(0, 0, True, False)
    step(1, 1, True, False)
    lax.fori_loop(1, TPW // 2 - 1, body, 0, unroll=False)
    step(TPW - 3, 0, False, False)
    step(TPW - 2, 1, False, True)
    step(TPW - 1, 0, False, True)
    wait_out(0)
    wait_out(1)


def _emb_kernel(indices_hbm, offsets_hbm, tables_hbm, out_hbm,
                idxraw, idx2, rows, acc, sem_g):
    del offsets_hbm
    wid = lax.axis_index("c") * NS + lax.axis_index("s")
    base = wid * BW

    def do_table(t, _):
        off = t * (B * G) + base * G
        pltpu.sync_copy(indices_hbm.at[pl.ds(off, R)], idxraw)
        tv = t * V

        def adj(j, _):
            for c in range(128 // 16):
                idx2[j, pl.ds(c * 16, 16)] = (
                    idxraw[pl.ds(j * 128 + c * 16, 16)] + tv)
            return 0

        lax.fori_loop(0, IC, adj, 0, unroll=False)

        copies = [
            pltpu.async_copy(tables_hbm.at[idx2.at[j]],
                             rows.at[pl.ds(j * 128, 128)], sem_g)
            for j in range(IC)
        ]
        for c in copies:
            c.wait()

        def bag(b, _):
            r0 = b * G
            s = rows[r0, :]
            for g in range(1, G):
                s = s + rows[r0 + g, :]
            acc[b, pl.ds(t * D, D)] = s
            return 0

        lax.fori_loop(0, BW, bag, 0, unroll=False)
        return 0

    lax.fori_loop(0, T, do_table, 0, unroll=False)
    pltpu.sync_copy(acc, out_hbm.at[pl.ds(base, BW)])


def kernel(indices, offsets, tables):
    tdv_flat = jnp.transpose(tables, (0, 2, 1)).reshape(-1)

    mesh = plsc.VectorSubcoreMesh(
        core_axis_name="c", subcore_axis_name="s",
        num_cores=NC, num_subcores=NS)
    sc_params = pltpu.CompilerParams(use_tc_tiling_on_sc=False)

    tvd_flat = functools.partial(
        pl.kernel,
        out_type=jax.ShapeDtypeStruct((T * V * D,), jnp.float32),
        mesh=mesh,
        scratch_types=[
            pltpu.VMEM((D * LIN,), jnp.float32),
            pltpu.VMEM((D * LIN,), jnp.float32),
            pltpu.VMEM((TC_ * D,), jnp.float32),
            pltpu.VMEM((TC_ * D,), jnp.float32),
            pltpu.SemaphoreType.DMA,
            pltpu.SemaphoreType.DMA,
            pltpu.SemaphoreType.DMA,
            pltpu.SemaphoreType.DMA,
        ],
        compiler_params=pltpu.CompilerParams(
            use_tc_tiling_on_sc=False, needs_layout_passes=False),
    )(_tr_kernel)(tdv_flat)

    run = functools.partial(
        pl.kernel,
        out_type=jax.ShapeDtypeStruct((B, T * D), jnp.float32),
        mesh=mesh,
        scratch_types=[
            pltpu.VMEM((R,), jnp.int32),
            pltpu.VMEM((IC, 128), jnp.int32),
            pltpu.VMEM((R, D), jnp.float32),
            pltpu.VMEM((BW, T * D), jnp.float32),
            pltpu.SemaphoreType.DMA,
        ],
        compiler_params=sc_params,
    )(_emb_kernel)
    return run(indices, offsets, tvd_flat.reshape(T * V, D))


# speedup vs baseline: 1.5080x; 1.5080x over previous
import functools

import jax
import jax.numpy as jnp
from jax import lax
from jax.experimental import pallas as pl
from jax.experimental.pallas import tpu as pltpu
from jax.experimental.pallas import tpu_sc as plsc

T = 26
V = 100000
D = 16
B = 4096
G = 20

NC, NS = 2, 16
NW = NC * NS
BW = B // NW
R = BW * G
IC = R // 128

TC_ = 1250
LIN = TC_ + 8 - 2
NCH = V // TC_
NTASK = T * NCH
TPW = NTASK // NW


def _tr_kernel(src_hbm, dst_hbm, in0, in1, out0, out1,
               sin0, sin1, sout0, sout1):
    wid = lax.axis_index("c") * NS + lax.axis_index("s")
    inb, outb, sin, sout = (in0, in1), (out0, out1), (sin0, sin1), (sout0, sout1)

    def task_pos(k):
        tsk = wid * TPW + k
        t = tsk // NCH
        v0 = (tsk % NCH) * TC_
        a0 = (v0 // 8) * 8
        return t, v0, a0

    def fire_in(k, p):
        t, _, a0 = task_pos(k)
        for d in range(D):
            pltpu.async_copy(
                src_hbm.at[pl.ds(t * (V * D) + d * V + a0, LIN)],
                inb[p].at[pl.ds(d * LIN, LIN)], sin[p])

    def wait_in(p):
        for d in range(D):
            pltpu.make_async_copy(
                src_hbm.at[pl.ds(d * LIN, LIN)],
                inb[p].at[pl.ds(d * LIN, LIN)], sin[p]).wait()

    def wait_out(p):
        pltpu.make_async_copy(
            outb[p], dst_hbm.at[pl.ds(0, TC_ * D)], sout[p]).wait()

    def step(k, p, first, last):
        wait_in(p)
        if not first:
            wait_out(p)
        t, v0, a0 = task_pos(k)
        m = v0 - a0
        lanes = lax.iota(jnp.int32, D)

        idx0 = lax.iota(jnp.int32, D) * LIN + m
        K = 10

        def rows(j, idx):
            gs = [plsc.load_gather(inb[p], [idx + jj]) for jj in range(K)]
            v0b = j * (K * D)
            for jj in range(K):
                outb[p][pl.ds(v0b + jj * D, D)] = gs[jj]
            return idx + K

        lax.fori_loop(0, TC_ // K, rows, idx0, unroll=2)
        pltpu.async_copy(
            outb[p], dst_hbm.at[pl.ds(t * (V * D) + v0 * D, TC_ * D)],
            sout[p])
        if not last:
            fire_in(k + 2, p)

    fire_in(0, 0)
    fire_in(1, 1)

    def body(j, _):
        step(2 * j, 0, False, False)
        step(2 * j + 1, 1, False, False)
        return 0

    step(0, 0, True, False)
    step(1, 1, True, False)
    lax.fori_loop(1, TPW // 2 - 1, body, 0, unroll=False)
    step(TPW - 3, 0, False, False)
    step(TPW - 2, 1, False, True)
    step(TPW - 1, 0, False, True)
    wait_out(0)
    wait_out(1)


def _emb_kernel(indices_hbm, offsets_hbm, tables_hbm, out_hbm,
                idxraw, idx2, rows, acc, sem_g):
    del offsets_hbm
    wid = lax.axis_index("c") * NS + lax.axis_index("s")
    base = wid * BW

    def do_table(t, _):
        off = t * (B * G) + base * G
        pltpu.sync_copy(indices_hbm.at[pl.ds(off, R)], idxraw)
        tv = t * V

        def adj(j, _):
            for c in range(128 // 16):
                idx2[j, pl.ds(c * 16, 16)] = (
                    idxraw[pl.ds(j * 128 + c * 16, 16)] + tv)
            return 0

        lax.fori_loop(0, IC, adj, 0, unroll=False)

        copies = [
            pltpu.async_copy(tables_hbm.at[idx2.at[j]],
                             rows.at[pl.ds(j * 128, 128)], sem_g)
            for j in range(IC)
        ]
        for c in copies:
            c.wait()

        def bag(b, _):
            r0 = b * G
            s = rows[r0, :]
            for g in range(1, G):
                s = s + rows[r0 + g, :]
            acc[b, pl.ds(t * D, D)] = s
            return 0

        lax.fori_loop(0, BW, bag, 0, unroll=False)
        return 0

    lax.fori_loop(0, T, do_table, 0, unroll=False)
    pltpu.sync_copy(acc, out_hbm.at[pl.ds(base, BW)])


def kernel(indices, offsets, tables):
    tdv_flat = jnp.transpose(tables, (0, 2, 1)).reshape(-1)

    mesh = plsc.VectorSubcoreMesh(
        core_axis_name="c", subcore_axis_name="s",
        num_cores=NC, num_subcores=NS)
    sc_params = pltpu.CompilerParams(use_tc_tiling_on_sc=False)

    tvd_flat = functools.partial(
        pl.kernel,
        out_type=jax.ShapeDtypeStruct((T * V * D,), jnp.float32),
        mesh=mesh,
        scratch_types=[
            pltpu.VMEM((D * LIN,), jnp.float32),
            pltpu.VMEM((D * LIN,), jnp.float32),
            pltpu.VMEM((TC_ * D,), jnp.float32),
            pltpu.VMEM((TC_ * D,), jnp.float32),
            pltpu.SemaphoreType.DMA,
            pltpu.SemaphoreType.DMA,
            pltpu.SemaphoreType.DMA,
            pltpu.SemaphoreType.DMA,
        ],
        compiler_params=pltpu.CompilerParams(
            use_tc_tiling_on_sc=False, needs_layout_passes=False),
    )(_tr_kernel)(tdv_flat)

    run = functools.partial(
        pl.kernel,
        out_type=jax.ShapeDtypeStruct((B, T * D), jnp.float32),
        mesh=mesh,
        scratch_types=[
            pltpu.VMEM((R,), jnp.int32),
            pltpu.VMEM((IC, 128), jnp.int32),
            pltpu.VMEM((R, D), jnp.float32),
            pltpu.VMEM((BW, T * D), jnp.float32),
            pltpu.SemaphoreType.DMA,
        ],
        compiler_params=sc_params,
    )(_emb_kernel)
    return run(indices, offsets, tvd_flat.reshape(T * V, D))


# speedup vs baseline: 1.7836x; 1.1828x over previous
import functools

import jax
import jax.numpy as jnp
from jax import lax
from jax.experimental import pallas as pl
from jax.experimental.pallas import tpu as pltpu
from jax.experimental.pallas import tpu_sc as plsc

T = 26
V = 100000
D = 16
B = 4096
G = 20

NC, NS = 2, 16
NW = NC * NS
BW = B // NW
R = BW * G
IC = R // 128

TC_ = 1250
LIN = TC_ + 8 - 2
NCH = V // TC_
NTASK = T * NCH
TPW = NTASK // NW


def _tr_kernel(src_hbm, dst_hbm, in0, in1, out0, out1,
               sin0, sin1, sout0, sout1):
    wid = lax.axis_index("c") * NS + lax.axis_index("s")
    inb, outb, sin, sout = (in0, in1), (out0, out1), (sin0, sin1), (sout0, sout1)

    def task_pos(k):
        tsk = wid * TPW + k
        t = tsk // NCH
        v0 = (tsk % NCH) * TC_
        a0 = (v0 // 8) * 8
        return t, v0, a0

    def fire_in(k, p):
        t, _, a0 = task_pos(k)
        for d in range(D):
            pltpu.async_copy(
                src_hbm.at[pl.ds(t * (V * D) + d * V + a0, LIN)],
                inb[p].at[pl.ds(d * LIN, LIN)], sin[p])

    def wait_in(p):
        for d in range(D):
            pltpu.make_async_copy(
                src_hbm.at[pl.ds(d * LIN, LIN)],
                inb[p].at[pl.ds(d * LIN, LIN)], sin[p]).wait()

    def wait_out(p):
        pltpu.make_async_copy(
            outb[p], dst_hbm.at[pl.ds(0, TC_ * D)], sout[p]).wait()

    def step(k, p, first, last):
        wait_in(p)
        if not first:
            wait_out(p)
        t, v0, a0 = task_pos(k)
        m = v0 - a0
        lanes = lax.iota(jnp.int32, D)

        idx0 = lax.iota(jnp.int32, D) * LIN + m
        K = 10

        def rows(j, idx):
            gs = [plsc.load_gather(inb[p], [idx + jj]) for jj in range(K)]
            v0b = j * (K * D)
            for jj in range(K):
                outb[p][pl.ds(v0b + jj * D, D)] = gs[jj]
            return idx + K

        lax.fori_loop(0, TC_ // K, rows, idx0, unroll=2)
        pltpu.async_copy(
            outb[p], dst_hbm.at[pl.ds(t * (V * D) + v0 * D, TC_ * D)],
            sout[p])
        if not last:
            fire_in(k + 2, p)

    fire_in(0, 0)
    fire_in(1, 1)

    def body(j, _):
        step(2 * j, 0, False, False)
        step(2 * j + 1, 1, False, False)
        return 0

    step(0, 0, True, False)
    step(1, 1, True, False)
    lax.fori_loop(1, TPW // 2 - 1, body, 0, unroll=False)
    step(TPW - 3, 0, False, False)
    step(TPW - 2, 1, False, True)
    step(TPW - 1, 0, False, True)
    wait_out(0)
    wait_out(1)


HB = BW // 2
HR = HB * G
HI = HR // 128
NST = T * 2


def _emb_kernel(indices_hbm, offsets_hbm, tables_hbm, out_hbm,
                ir0, ir1, ix0, ix1, rw0, rw1, acc,
                si0, si1, sg0, sg1):
    del offsets_hbm
    wid = lax.axis_index("c") * NS + lax.axis_index("s")
    base = wid * BW
    idxraw, idx2, rows = (ir0, ir1), (ix0, ix1), (rw0, rw1)
    si, sg = (si0, si1), (sg0, sg1)

    def fire_idx(s, p):
        off = (s // 2) * (B * G) + (base + (s % 2) * HB) * G
        pltpu.async_copy(indices_hbm.at[pl.ds(off, HR)], idxraw[p], si[p])

    def wait_idx(p):
        pltpu.make_async_copy(
            indices_hbm.at[pl.ds(0, HR)], idxraw[p], si[p]).wait()

    def bias_and_fire_rows(s, p):
        tv = (s // 2) * V

        def adj(j, _):
            for c in range(128 // 16):
                idx2[p][j, pl.ds(c * 16, 16)] = (
                    idxraw[p][pl.ds(j * 128 + c * 16, 16)] + tv)
            return 0

        lax.fori_loop(0, HI, adj, 0, unroll=True)
        for j in range(HI):
            pltpu.async_copy(tables_hbm.at[idx2[p].at[j]],
                             rows[p].at[pl.ds(j * 128, 128)], sg[p])

    def wait_rows(p):
        for j in range(HI):
            pltpu.make_async_copy(
                tables_hbm.at[pl.ds(0, 128)],
                rows[p].at[pl.ds(j * 128, 128)], sg[p]).wait()

    def bagsum(s, p):
        t = s // 2
        b0 = (s % 2) * HB

        def bag(b, _):
            r0 = b * G
            v = rows[p][r0, :]
            for g in range(1, G):
                v = v + rows[p][r0 + g, :]
            acc[b0 + b, pl.ds(t * D, D)] = v
            return 0

        lax.fori_loop(0, HB, bag, 0, unroll=False)

    fire_idx(0, 0)
    fire_idx(1, 1)
    wait_idx(0)
    bias_and_fire_rows(0, 0)
    fire_idx(2, 0)

    def body(i, _):
        for p in (1, 0):
            s = 2 * i + 2 - p
            wait_idx(p)
            bias_and_fire_rows(s, p)
            fire_idx(jnp.minimum(s + 2, NST - 1), p)
            wait_rows(1 - p)
            bagsum(s - 1, 1 - p)
        return 0

    lax.fori_loop(0, (NST - 2) // 2, body, 0, unroll=False)
    p = 1
    s = NST - 1
    wait_idx(p)
    bias_and_fire_rows(s, p)
    wait_rows(1 - p)
    bagsum(s - 1, 1 - p)
    wait_rows(p)
    bagsum(s, p)
    wait_idx(0)
    pltpu.sync_copy(acc, out_hbm.at[pl.ds(base, BW)])


def kernel(indices, offsets, tables):
    tdv_flat = jnp.transpose(tables, (0, 2, 1)).reshape(-1)

    mesh = plsc.VectorSubcoreMesh(
        core_axis_name="c", subcore_axis_name="s",
        num_cores=NC, num_subcores=NS)
    sc_params = pltpu.CompilerParams(use_tc_tiling_on_sc=False)

    tvd_flat = functools.partial(
        pl.kernel,
        out_type=jax.ShapeDtypeStruct((T * V * D,), jnp.float32),
        mesh=mesh,
        scratch_types=[
            pltpu.VMEM((D * LIN,), jnp.float32),
            pltpu.VMEM((D * LIN,), jnp.float32),
            pltpu.VMEM((TC_ * D,), jnp.float32),
            pltpu.VMEM((TC_ * D,), jnp.float32),
            pltpu.SemaphoreType.DMA,
            pltpu.SemaphoreType.DMA,
            pltpu.SemaphoreType.DMA,
            pltpu.SemaphoreType.DMA,
        ],
        compiler_params=pltpu.CompilerParams(
            use_tc_tiling_on_sc=False, needs_layout_passes=False),
    )(_tr_kernel)(tdv_flat)

    run = functools.partial(
        pl.kernel,
        out_type=jax.ShapeDtypeStruct((B, T * D), jnp.float32),
        mesh=mesh,
        scratch_types=[
            pltpu.VMEM((HR,), jnp.int32),
            pltpu.VMEM((HR,), jnp.int32),
            pltpu.VMEM((HI, 128), jnp.int32),
            pltpu.VMEM((HI, 128), jnp.int32),
            pltpu.VMEM((HR, D), jnp.float32),
            pltpu.VMEM((HR, D), jnp.float32),
            pltpu.VMEM((BW, T * D), jnp.float32),
            pltpu.SemaphoreType.DMA,
            pltpu.SemaphoreType.DMA,
            pltpu.SemaphoreType.DMA,
            pltpu.SemaphoreType.DMA,
        ],
        compiler_params=sc_params,
    )(_emb_kernel)
    return run(indices, offsets, tvd_flat.reshape(T * V, D))
